# baseline (speedup 1.0000x reference)
import functools

import jax
import jax.numpy as jnp
from jax import lax
from jax.experimental import pallas as pl
from jax.experimental.pallas import tpu as pltpu
from jax.experimental.pallas import tpu_sc as plsc

N_USERS = 100000
N_ITEMS = 100000
IN_DIM = 384
BRANCH_DIM = 32
TEMPERATURE = 0.2
NEG_PER_POS = 64
NUM_CAND = 512

N_NODES = N_USERS + N_ITEMS
NNZ = 6400000
D = BRANCH_DIM
NCORE, NSUB = 2, 16
CS = 34560
CS_TOT = CS + 256
EPT = NNZ // NSUB
BE = 1600
NG = BE // 16
GSUB = 128
WPT = CS // NSUB


def _gcn_layer_kernel(table_h, row_h, col_h, val_h, out_h, row_va, col_va,
                      val_va, row_vb, col_vb, val_vb, colc, rowc, valc,
                      rowc_ga, colc_ga, rows_va, rowc_gb, colc_gb, rows_vb,
                      rowc_gc, colc_gc, rows_vc, rowc_gd, colc_gd, rows_vd,
                      zbuf, acc, sema, semb, semc, semd, esema, esemb):
    cid = lax.axis_index("c")
    sid = lax.axis_index("s")
    lane = lax.iota(jnp.int32, 16)
    zv = jnp.zeros((16,), jnp.float32)
    NB = EPT // BE
    sets = ((row_va, col_va, val_va, esema), (row_vb, col_vb, val_vb, esemb))

    def zfill(i, _):
        zbuf[i, pl.ds(0, 16)] = zv
        zbuf[i, pl.ds(16, 16)] = zv
        return 0

    lax.fori_loop(0, 64, zfill, 0)
    ebase = sid * EPT

    def _fire(b, st):
        eb = ebase + jnp.minimum(b, NB - 1) * BE
        pltpu.async_copy(row_h.at[pl.ds(eb, BE)], st[0], st[3])
        pltpu.async_copy(col_h.at[pl.ds(eb, BE)], st[1], st[3])
        pltpu.async_copy(val_h.at[pl.ds(eb, BE)], st[2], st[3])

    def _wait(st):
        for ref in st[:3]:
            pltpu.make_async_copy(row_h.at[pl.ds(0, BE)], ref, st[3]).wait()

    for chunk in range(3):
        ck = cid * 3 + chunk
        r0 = ck * CS
        zb = sid * (CS_TOT // NSUB)

        def zcp(z, _):
            pltpu.sync_copy(zbuf, acc.at[pl.ds(zb + z * 64, 64)])
            return 0

        lax.fori_loop(0, CS_TOT // NSUB // 64, zcp, 0)
        plsc.subcore_barrier()

        _fire(0, sets[0])
        _fire(1, sets[1])

        gsets = ((colc_ga, rowc_ga, rows_va, sema),
                 (colc_gb, rowc_gb, rows_vb, semb),
                 (colc_gc, rowc_gc, rows_vc, semc),
                 (colc_gd, rowc_gd, rows_vd, semd))

        def gpair(jj, _):
            descs = []
            for half in range(4):
                cg, rg, rv_, sm = gsets[half]
                off = jj * 4 * GSUB + half * GSUB

                def cpg(t, _):
                    cg[pl.ds(t * 16, 16)] = colc[pl.ds(off + t * 16, 16)]
                    rg[pl.ds(t * 16, 16)] = rowc[pl.ds(off + t * 16, 16)]
                    return 0

                lax.fori_loop(0, GSUB // 16, cpg, 0)
                descs.append(pltpu.async_copy(table_h.at[cg], rv_, sm))
            sdescs = []
            for half in range(4):
                cg, rg, rv_, sm = gsets[half]
                off = jj * 4 * GSUB + half * GSUB
                descs[half].wait()

                def srow16(t, _):
                    vv = valc[pl.ds(off + t * 16, 16)]
                    for q in range(16):
                        r = t * 16 + q
                        v = vv[q]
                        rv_[r, pl.ds(0, 16)] = rv_[r, pl.ds(0, 16)] * v
                        rv_[r, pl.ds(16, 16)] = rv_[r, pl.ds(16, 16)] * v
                    return 0

                lax.fori_loop(0, GSUB // 16, srow16, 0)
                sdescs.append(pltpu.async_copy(rv_, acc.at[rg], sm, add=True))
            for d in sdescs:
                d.wait()
            return 0

        def pair_body(pb, s):
            for par in range(2):
                st = sets[par]
                b = pb * 2 + par
                _wait(st)
                row_v, col_v, val_v = st[0], st[1], st[2]

                def grp4(g4, sc):
                    for u in range(20):
                        g = g4 * 20 + u
                        rv = row_v[pl.ds(g * 16, 16)]
                        cv = col_v[pl.ds(g * 16, 16)]
                        vv = val_v[pl.ds(g * 16, 16)]
                        m = (rv >= r0) & (rv < r0 + CS)
                        c = plsc.cumsum(m.astype(jnp.int32))
                        pos = sc + c - 1
                        plsc.store_scatter(colc, [pos], cv, mask=m)
                        plsc.store_scatter(rowc, [pos], rv - r0, mask=m)
                        plsc.store_scatter(valc, [pos], vv, mask=m)
                        sc = sc + c[15]
                    return sc

                s = lax.fori_loop(0, NG // 20, grp4, s)
                _fire(b + 2, st)

                nproc = (s // (4 * GSUB)) * (4 * GSUB)
                lax.fori_loop(0, nproc // (4 * GSUB), gpair, 0)

                def lcp(t, _):
                    colc[pl.ds(t * 16, 16)] = colc[pl.ds(nproc + t * 16, 16)]
                    rowc[pl.ds(t * 16, 16)] = rowc[pl.ds(nproc + t * 16, 16)]
                    valc[pl.ds(t * 16, 16)] = valc[pl.ds(nproc + t * 16, 16)]
                    return 0

                lax.fori_loop(0, 4 * GSUB // 16, lcp, 0)
                s = s - nproc
            return s

        s_left = lax.fori_loop(0, NB // 2, pair_body, 0)

        s_pad = ((s_left + 4 * GSUB - 1) // (4 * GSUB)) * (4 * GSUB)

        def padb(_, sc):
            colc[pl.ds(sc, 16)] = sid * 12345 + lane * 677
            rowc[pl.ds(sc, 16)] = CS + sid * 16 + lane
            valc[pl.ds(sc, 16)] = zv
            return sc + 16

        lax.fori_loop(0, (s_pad - s_left + 15) // 16, padb, s_left)
        lax.fori_loop(0, s_pad // (4 * GSUB), gpair, 0)
        _wait(sets[0])
        _wait(sets[1])
        plsc.subcore_barrier()
        g0 = r0 + sid * WPT

        @pl.when(g0 + WPT <= N_NODES)
        def _():
            pltpu.sync_copy(acc.at[pl.ds(sid * WPT, WPT)],
                            out_h.at[pl.ds(g0, WPT)])

        @pl.when(g0 == 5 * CS + 12 * WPT)
        def _():
            pltpu.sync_copy(acc.at[pl.ds(sid * WPT, N_NODES - 5 * CS - 12 * WPT)],
                            out_h.at[pl.ds(g0, N_NODES - 5 * CS - 12 * WPT)])

        plsc.subcore_barrier()


def _gcn_layer(table, row, col, val):
    mesh = plsc.VectorSubcoreMesh(core_axis_name="c", subcore_axis_name="s")
    kfn = functools.partial(
        pl.kernel,
        out_type=jax.ShapeDtypeStruct((N_NODES, D), jnp.float32),
        mesh=mesh,
        compiler_params=pltpu.CompilerParams(use_tc_tiling_on_sc=False,
                                             needs_layout_passes=False),
        scratch_types=[
            pltpu.VMEM((BE,), jnp.int32),
            pltpu.VMEM((BE,), jnp.int32),
            pltpu.VMEM((BE,), jnp.float32),
            pltpu.VMEM((BE,), jnp.int32),
            pltpu.VMEM((BE,), jnp.int32),
            pltpu.VMEM((BE,), jnp.float32),
            pltpu.VMEM((BE + 5 * GSUB,), jnp.int32),
            pltpu.VMEM((BE + 5 * GSUB,), jnp.int32),
            pltpu.VMEM((BE + 5 * GSUB,), jnp.float32),
            pltpu.VMEM((GSUB,), jnp.int32),
            pltpu.VMEM((GSUB,), jnp.int32),
            pltpu.VMEM((GSUB, D), jnp.float32),
            pltpu.VMEM((GSUB,), jnp.int32),
            pltpu.VMEM((GSUB,), jnp.int32),
            pltpu.VMEM((GSUB, D), jnp.float32),
            pltpu.VMEM((GSUB,), jnp.int32),
            pltpu.VMEM((GSUB,), jnp.int32),
            pltpu.VMEM((GSUB, D), jnp.float32),
            pltpu.VMEM((GSUB,), jnp.int32),
            pltpu.VMEM((GSUB,), jnp.int32),
            pltpu.VMEM((GSUB, D), jnp.float32),
            pltpu.VMEM((64, D), jnp.float32),
            pltpu.VMEM_SHARED((CS_TOT, D), jnp.float32),
            pltpu.SemaphoreType.DMA,
            pltpu.SemaphoreType.DMA,
            pltpu.SemaphoreType.DMA,
            pltpu.SemaphoreType.DMA,
            pltpu.SemaphoreType.DMA,
            pltpu.SemaphoreType.DMA,
        ],
    )(_gcn_layer_kernel)
    return kfn(table, row, col, val)


FD = 2 * BRANCH_DIM
SB = 1024 // (NCORE * NSUB)


def _score_kernel(alli_h, bu_h, cand_h, pos_h, out_h, pout_h, cidx_v, cidx_g,
                  rows_v, bu_v, scores_v, posidx_g, posrow_v, pos_sv, sem):
    cid = lax.axis_index("c")
    sid = lax.axis_index("s")
    wid = sid * NCORE + cid
    lane = lax.iota(jnp.int32, 16)
    b0 = wid * SB
    pltpu.sync_copy(bu_h.at[pl.ds(b0, SB)], bu_v)
    pltpu.sync_copy(pos_h.at[pl.ds(b0, SB)], posidx_g)
    pltpu.async_copy(alli_h.at[posidx_g], posrow_v, sem).wait()

    for h in range(2):

        def row16(i, pvec):
            bi = h * 16 + i
            u0 = bu_v[bi, pl.ds(0, 16)]
            u1 = bu_v[bi, pl.ds(16, 16)]
            u2 = bu_v[bi, pl.ds(32, 16)]
            u3 = bu_v[bi, pl.ds(48, 16)]
            pltpu.sync_copy(
                cand_h.at[pl.ds((b0 + bi) * NUM_CAND, NUM_CAND)], cidx_v)

            def sub(j, _):
                off = j * GSUB

                def cpg(t, _):
                    cidx_g[pl.ds(t * 16, 16)] = cidx_v[pl.ds(off + t * 16, 16)]
                    return 0

                lax.fori_loop(0, GSUB // 16, cpg, 0)
                pltpu.async_copy(alli_h.at[cidx_g], rows_v, sem).wait()

                def s16(t, _):
                    sv = jnp.zeros((16,), jnp.float32)
                    for q in range(16):
                        r = t * 16 + q
                        acc = (rows_v[r, pl.ds(0, 16)] * u0
                               + rows_v[r, pl.ds(16, 16)] * u1
                               + rows_v[r, pl.ds(32, 16)] * u2
                               + rows_v[r, pl.ds(48, 16)] * u3)
                        sv = jnp.where(lane == q, jnp.sum(acc), sv)
                    scores_v[pl.ds(off + t * 16, 16)] = sv
                    return 0

                lax.fori_loop(0, GSUB // 16, s16, 0)
                return 0

            lax.fori_loop(0, NUM_CAND // GSUB, sub, 0)
            pltpu.sync_copy(scores_v,
                            out_h.at[pl.ds((b0 + bi) * NUM_CAND, NUM_CAND)])
            pacc = (posrow_v[bi, pl.ds(0, 16)] * u0
                    + posrow_v[bi, pl.ds(16, 16)] * u1
                    + posrow_v[bi, pl.ds(32, 16)] * u2
                    + posrow_v[bi, pl.ds(48, 16)] * u3)
            return jnp.where(lane == i, jnp.sum(pacc), pvec)

        pvec = lax.fori_loop(0, 16, row16, jnp.zeros((16,), jnp.float32))
        pos_sv[...] = pvec
        pltpu.sync_copy(pos_sv, pout_h.at[pl.ds(b0 + h * 16, 16)])


def _score_call(all_i, batch_u, cand_flat, pos_iids):
    mesh = plsc.VectorSubcoreMesh(core_axis_name="c", subcore_axis_name="s")
    kfn = functools.partial(
        pl.kernel,
        out_type=(jax.ShapeDtypeStruct((1024 * NUM_CAND,), jnp.float32),
                  jax.ShapeDtypeStruct((1024,), jnp.float32)),
        mesh=mesh,
        compiler_params=pltpu.CompilerParams(use_tc_tiling_on_sc=False,
                                             needs_layout_passes=False),
        scratch_types=[
            pltpu.VMEM((NUM_CAND,), jnp.int32),
            pltpu.VMEM((GSUB,), jnp.int32),
            pltpu.VMEM((GSUB, FD), jnp.float32),
            pltpu.VMEM((SB, FD), jnp.float32),
            pltpu.VMEM((NUM_CAND,), jnp.float32),
            pltpu.VMEM((SB,), jnp.int32),
            pltpu.VMEM((SB, FD), jnp.float32),
            pltpu.VMEM((16,), jnp.float32),
            pltpu.SemaphoreType.DMA,
        ],
    )(_score_kernel)
    return kfn(all_i, batch_u, cand_flat, pos_iids)


def _loss_kernel(s_ref, p_ref, o_ref):
    s = s_ref[...]
    p = p_ref[...]
    v = jax.lax.bitcast_convert_type(s, jnp.uint32)
    sign = (v >> 31).astype(jnp.uint32)
    key = v ^ jnp.where(sign == 1, jnp.uint32(0xFFFFFFFF),
                        jnp.uint32(0x80000000))

    def bit_step(i, lo):
        cand = lo | (jnp.uint32(1) << (31 - i))
        cnt = jnp.sum((key >= cand[:, None]).astype(jnp.int32), axis=1)
        return jnp.where(cnt >= NEG_PER_POS, cand, lo)

    lo = lax.fori_loop(0, 32, bit_step,
                       jnp.zeros((s.shape[0],), jnp.uint32))
    t = lo[:, None]
    gt = key > t
    c_gt = jnp.sum(gt.astype(jnp.int32), axis=1, keepdims=True)
    s_t = jnp.max(jnp.where(key <= t, s, -jnp.inf), axis=1, keepdims=True)
    smax = jnp.max(s, axis=1, keepdims=True)
    pli = p / TEMPERATURE
    m = jnp.maximum(pli, smax / TEMPERATURE)
    sumexp = (jnp.exp(pli - m)
              + jnp.sum(jnp.where(gt, jnp.exp(s / TEMPERATURE - m), 0.0),
                        axis=1, keepdims=True)
              + (NEG_PER_POS - c_gt).astype(jnp.float32)
              * jnp.exp(s_t / TEMPERATURE - m))
    lse = m + jnp.log(sumexp)
    o_ref[...] = jnp.mean(lse - pli).reshape(1, 1)


def _loss_call(scores, pos_scores):
    return pl.pallas_call(
        _loss_kernel,
        in_specs=[pl.BlockSpec((1024, NUM_CAND), lambda: (0, 0)),
                  pl.BlockSpec((1024, 1), lambda: (0, 0))],
        out_specs=pl.BlockSpec((1, 1), lambda: (0, 0)),
        out_shape=jax.ShapeDtypeStruct((1, 1), jnp.float32),
    )(scores, pos_scores)


def _normalize(x, axis=-1):
    n = jnp.linalg.norm(x, axis=axis, keepdims=True)
    return x / jnp.maximum(n, 1e-12)


def _sem_matmul_norm_block(x_ref, w_ref, b_ref, o_ref):
    y = jnp.dot(x_ref[...], w_ref[...], preferred_element_type=jnp.float32)
    y = y + b_ref[...]
    n = jnp.sqrt(jnp.sum(y * y, axis=-1, keepdims=True))
    o_ref[...] = y / jnp.maximum(n, 1e-12)


def _sem_matmul_norm(x, W, b):
    M = x.shape[0]
    BM = 2000
    assert M % BM == 0
    return pl.pallas_call(
        _sem_matmul_norm_block,
        grid=(M // BM,),
        in_specs=[
            pl.BlockSpec((BM, IN_DIM), lambda i: (i, 0)),
            pl.BlockSpec((IN_DIM, BRANCH_DIM), lambda i: (0, 0)),
            pl.BlockSpec((1, BRANCH_DIM), lambda i: (0, 0)),
        ],
        out_specs=pl.BlockSpec((BM, BRANCH_DIM), lambda i: (i, 0)),
        out_shape=jax.ShapeDtypeStruct((M, BRANCH_DIM), jnp.float32),
    )(x, W, b)


def kernel(raw_item_embs, user_sem_base, sem_W, sem_b, collab_user_emb,
           collab_item_emb, adj_val, adj_row, adj_col, uids, pos_iids,
           neg_per_pos, hard_neg_factor):
    b2d = sem_b.reshape(1, BRANCH_DIM)
    i_sem = _sem_matmul_norm(raw_item_embs, sem_W, b2d)
    u_sem_batch = _normalize(user_sem_base[uids] @ sem_W + sem_b)

    ego = jnp.concatenate([collab_user_emb, collab_item_emb], axis=0)
    ego1 = _gcn_layer(ego, adj_row, adj_col, adj_val)
    ego2 = _gcn_layer(ego1, adj_row, adj_col, adj_val)
    final = (ego + ego1 + ego2) / 3.0
    u_collab_batch = _normalize(final[uids])
    i_collab = _normalize(final[N_USERS:])

    all_i = _normalize(jnp.concatenate([i_collab, i_sem], axis=1))
    batch_u = _normalize(jnp.concatenate([u_collab_batch, u_sem_batch], axis=1))

    B = uids.shape[0]
    cand_ids = jax.random.randint(jax.random.key(42), (B, NUM_CAND), 0, N_ITEMS,
                                  dtype=jnp.int32)
    cand_ids = jnp.where(cand_ids == pos_iids[:, None],
                         (cand_ids + 1) % N_ITEMS, cand_ids)
    scores_flat, pos_scores = _score_call(all_i, batch_u,
                                          cand_ids.reshape(-1), pos_iids)
    scores = scores_flat.reshape(B, NUM_CAND)
    loss = _loss_call(scores, pos_scores.reshape(B, 1)).reshape(())
    return loss + 0.0 * (neg_per_pos * hard_neg_factor - NUM_CAND)


# speedup vs baseline: 1.0257x; 1.0257x over previous
import functools

import jax
import jax.numpy as jnp
from jax import lax
from jax.experimental import pallas as pl
from jax.experimental.pallas import tpu as pltpu
from jax.experimental.pallas import tpu_sc as plsc

N_USERS = 100000
N_ITEMS = 100000
IN_DIM = 384
BRANCH_DIM = 32
TEMPERATURE = 0.2
NEG_PER_POS = 64
NUM_CAND = 512

N_NODES = N_USERS + N_ITEMS
NNZ = 6400000
D = BRANCH_DIM
NCORE, NSUB = 2, 16
CS = 34560
CS_TOT = CS + 256
EPT = NNZ // NSUB
BE = 1600
NG = BE // 16
GSUB = 128
WPT = CS // NSUB


def _gcn_layer_kernel(table_h, row_h, col_h, val_h, out_h, row_va, col_va,
                      val_va, row_vb, col_vb, val_vb, colc, rowc, valc,
                      rowc_ga, colc_ga, rows_va, rowc_gb, colc_gb, rows_vb,
                      rowc_gc, colc_gc, rows_vc, rowc_gd, colc_gd, rows_vd,
                      zbuf, acc, sema, semb, semc, semd, esema, esemb):
    cid = lax.axis_index("c")
    sid = lax.axis_index("s")
    lane = lax.iota(jnp.int32, 16)
    zv = jnp.zeros((16,), jnp.float32)
    NB = EPT // BE
    sets = ((row_va, col_va, val_va, esema), (row_vb, col_vb, val_vb, esemb))

    def zfill(i, _):
        zbuf[i, pl.ds(0, 16)] = zv
        zbuf[i, pl.ds(16, 16)] = zv
        return 0

    lax.fori_loop(0, 64, zfill, 0)
    ebase = sid * EPT

    def _fire(b, st):
        eb = ebase + jnp.minimum(b, NB - 1) * BE
        pltpu.async_copy(row_h.at[pl.ds(eb, BE)], st[0], st[3])
        pltpu.async_copy(col_h.at[pl.ds(eb, BE)], st[1], st[3])
        pltpu.async_copy(val_h.at[pl.ds(eb, BE)], st[2], st[3])

    def _wait(st):
        for ref in st[:3]:
            pltpu.make_async_copy(row_h.at[pl.ds(0, BE)], ref, st[3]).wait()

    for chunk in range(3):
        ck = cid * 3 + chunk
        r0 = ck * CS
        zb = sid * (CS_TOT // NSUB)

        def zcp(z, _):
            pltpu.sync_copy(zbuf, acc.at[pl.ds(zb + z * 64, 64)])
            return 0

        lax.fori_loop(0, CS_TOT // NSUB // 64, zcp, 0)
        plsc.subcore_barrier()

        _fire(0, sets[0])
        _fire(1, sets[1])

        gsets = ((colc_ga, rowc_ga, rows_va, sema),
                 (colc_gb, rowc_gb, rows_vb, semb),
                 (colc_gc, rowc_gc, rows_vc, semc),
                 (colc_gd, rowc_gd, rows_vd, semd))

        def gpair(jj, _):
            descs = []
            for half in range(4):
                cg, rg, rv_, sm = gsets[half]
                off = jj * 4 * GSUB + half * GSUB

                def cpg(t, _):
                    cg[pl.ds(t * 16, 16)] = colc[pl.ds(off + t * 16, 16)]
                    rg[pl.ds(t * 16, 16)] = rowc[pl.ds(off + t * 16, 16)]
                    return 0

                lax.fori_loop(0, GSUB // 16, cpg, 0)
                descs.append(pltpu.async_copy(table_h.at[cg], rv_, sm))
            sdescs = []
            for half in range(4):
                cg, rg, rv_, sm = gsets[half]
                off = jj * 4 * GSUB + half * GSUB
                descs[half].wait()

                def srow16(t, _):
                    vv = valc[pl.ds(off + t * 16, 16)]
                    for q in range(16):
                        r = t * 16 + q
                        v = vv[q]
                        rv_[r, pl.ds(0, 16)] = rv_[r, pl.ds(0, 16)] * v
                        rv_[r, pl.ds(16, 16)] = rv_[r, pl.ds(16, 16)] * v
                    return 0

                lax.fori_loop(0, GSUB // 16, srow16, 0)
                sdescs.append(pltpu.async_copy(rv_, acc.at[rg], sm, add=True))
            for d in sdescs:
                d.wait()
            return 0

        def pair_body(pb, s):
            for par in range(2):
                st = sets[par]
                b = pb * 2 + par
                _wait(st)
                row_v, col_v, val_v = st[0], st[1], st[2]

                def grp4(g4, sc):
                    for u in range(10):
                        g = g4 * 10 + u
                        rv = row_v[pl.ds(g * 16, 16)]
                        cv = col_v[pl.ds(g * 16, 16)]
                        vv = val_v[pl.ds(g * 16, 16)]
                        m = (rv >= r0) & (rv < r0 + CS)
                        c = plsc.cumsum(m.astype(jnp.int32))
                        pos = sc + c - 1
                        plsc.store_scatter(colc, [pos], cv, mask=m)
                        plsc.store_scatter(rowc, [pos], rv - r0, mask=m)
                        plsc.store_scatter(valc, [pos], vv, mask=m)
                        sc = sc + c[15]
                    return sc

                s = lax.fori_loop(0, NG // 10, grp4, s)
                _fire(b + 2, st)

                nproc = (s // (4 * GSUB)) * (4 * GSUB)
                lax.fori_loop(0, nproc // (4 * GSUB), gpair, 0)

                def lcp(t, _):
                    colc[pl.ds(t * 16, 16)] = colc[pl.ds(nproc + t * 16, 16)]
                    rowc[pl.ds(t * 16, 16)] = rowc[pl.ds(nproc + t * 16, 16)]
                    valc[pl.ds(t * 16, 16)] = valc[pl.ds(nproc + t * 16, 16)]
                    return 0

                lax.fori_loop(0, 4 * GSUB // 16, lcp, 0)
                s = s - nproc
            return s

        s_left = lax.fori_loop(0, NB // 2, pair_body, 0)

        s_pad = ((s_left + 4 * GSUB - 1) // (4 * GSUB)) * (4 * GSUB)

        def padb(_, sc):
            colc[pl.ds(sc, 16)] = sid * 12345 + lane * 677
            rowc[pl.ds(sc, 16)] = CS + sid * 16 + lane
            valc[pl.ds(sc, 16)] = zv
            return sc + 16

        lax.fori_loop(0, (s_pad - s_left + 15) // 16, padb, s_left)
        lax.fori_loop(0, s_pad // (4 * GSUB), gpair, 0)
        _wait(sets[0])
        _wait(sets[1])
        plsc.subcore_barrier()
        g0 = r0 + sid * WPT

        @pl.when(g0 + WPT <= N_NODES)
        def _():
            pltpu.sync_copy(acc.at[pl.ds(sid * WPT, WPT)],
                            out_h.at[pl.ds(g0, WPT)])

        @pl.when(g0 == 5 * CS + 12 * WPT)
        def _():
            pltpu.sync_copy(acc.at[pl.ds(sid * WPT, N_NODES - 5 * CS - 12 * WPT)],
                            out_h.at[pl.ds(g0, N_NODES - 5 * CS - 12 * WPT)])

        plsc.subcore_barrier()


def _gcn_layer(table, row, col, val):
    mesh = plsc.VectorSubcoreMesh(core_axis_name="c", subcore_axis_name="s")
    kfn = functools.partial(
        pl.kernel,
        out_type=jax.ShapeDtypeStruct((N_NODES, D), jnp.float32),
        mesh=mesh,
        compiler_params=pltpu.CompilerParams(use_tc_tiling_on_sc=False,
                                             needs_layout_passes=False),
        scratch_types=[
            pltpu.VMEM((BE,), jnp.int32),
            pltpu.VMEM((BE,), jnp.int32),
            pltpu.VMEM((BE,), jnp.float32),
            pltpu.VMEM((BE,), jnp.int32),
            pltpu.VMEM((BE,), jnp.int32),
            pltpu.VMEM((BE,), jnp.float32),
            pltpu.VMEM((BE + 5 * GSUB,), jnp.int32),
            pltpu.VMEM((BE + 5 * GSUB,), jnp.int32),
            pltpu.VMEM((BE + 5 * GSUB,), jnp.float32),
            pltpu.VMEM((GSUB,), jnp.int32),
            pltpu.VMEM((GSUB,), jnp.int32),
            pltpu.VMEM((GSUB, D), jnp.float32),
            pltpu.VMEM((GSUB,), jnp.int32),
            pltpu.VMEM((GSUB,), jnp.int32),
            pltpu.VMEM((GSUB, D), jnp.float32),
            pltpu.VMEM((GSUB,), jnp.int32),
            pltpu.VMEM((GSUB,), jnp.int32),
            pltpu.VMEM((GSUB, D), jnp.float32),
            pltpu.VMEM((GSUB,), jnp.int32),
            pltpu.VMEM((GSUB,), jnp.int32),
            pltpu.VMEM((GSUB, D), jnp.float32),
            pltpu.VMEM((64, D), jnp.float32),
            pltpu.VMEM_SHARED((CS_TOT, D), jnp.float32),
            pltpu.SemaphoreType.DMA,
            pltpu.SemaphoreType.DMA,
            pltpu.SemaphoreType.DMA,
            pltpu.SemaphoreType.DMA,
            pltpu.SemaphoreType.DMA,
            pltpu.SemaphoreType.DMA,
        ],
    )(_gcn_layer_kernel)
    return kfn(table, row, col, val)


FD = 2 * BRANCH_DIM
SB = 1024 // (NCORE * NSUB)


def _score_kernel(alli_h, bu_h, cand_h, pos_h, out_h, pout_h, cidx_v, cidx_g,
                  rows_v, bu_v, scores_v, posidx_g, posrow_v, pos_sv, sem):
    cid = lax.axis_index("c")
    sid = lax.axis_index("s")
    wid = sid * NCORE + cid
    lane = lax.iota(jnp.int32, 16)
    b0 = wid * SB
    pltpu.sync_copy(bu_h.at[pl.ds(b0, SB)], bu_v)
    pltpu.sync_copy(pos_h.at[pl.ds(b0, SB)], posidx_g)
    pltpu.async_copy(alli_h.at[posidx_g], posrow_v, sem).wait()

    for h in range(2):

        def row16(i, pvec):
            bi = h * 16 + i
            u0 = bu_v[bi, pl.ds(0, 16)]
            u1 = bu_v[bi, pl.ds(16, 16)]
            u2 = bu_v[bi, pl.ds(32, 16)]
            u3 = bu_v[bi, pl.ds(48, 16)]
            pltpu.sync_copy(
                cand_h.at[pl.ds((b0 + bi) * NUM_CAND, NUM_CAND)], cidx_v)

            def sub(j, _):
                off = j * GSUB

                def cpg(t, _):
                    cidx_g[pl.ds(t * 16, 16)] = cidx_v[pl.ds(off + t * 16, 16)]
                    return 0

                lax.fori_loop(0, GSUB // 16, cpg, 0)
                pltpu.async_copy(alli_h.at[cidx_g], rows_v, sem).wait()

                def s16(t, _):
                    sv = jnp.zeros((16,), jnp.float32)
                    for q in range(16):
                        r = t * 16 + q
                        acc = (rows_v[r, pl.ds(0, 16)] * u0
                               + rows_v[r, pl.ds(16, 16)] * u1
                               + rows_v[r, pl.ds(32, 16)] * u2
                               + rows_v[r, pl.ds(48, 16)] * u3)
                        sv = jnp.where(lane == q, jnp.sum(acc), sv)
                    scores_v[pl.ds(off + t * 16, 16)] = sv
                    return 0

                lax.fori_loop(0, GSUB // 16, s16, 0)
                return 0

            lax.fori_loop(0, NUM_CAND // GSUB, sub, 0)
            pltpu.sync_copy(scores_v,
                            out_h.at[pl.ds((b0 + bi) * NUM_CAND, NUM_CAND)])
            pacc = (posrow_v[bi, pl.ds(0, 16)] * u0
                    + posrow_v[bi, pl.ds(16, 16)] * u1
                    + posrow_v[bi, pl.ds(32, 16)] * u2
                    + posrow_v[bi, pl.ds(48, 16)] * u3)
            return jnp.where(lane == i, jnp.sum(pacc), pvec)

        pvec = lax.fori_loop(0, 16, row16, jnp.zeros((16,), jnp.float32))
        pos_sv[...] = pvec
        pltpu.sync_copy(pos_sv, pout_h.at[pl.ds(b0 + h * 16, 16)])


def _score_call(all_i, batch_u, cand_flat, pos_iids):
    mesh = plsc.VectorSubcoreMesh(core_axis_name="c", subcore_axis_name="s")
    kfn = functools.partial(
        pl.kernel,
        out_type=(jax.ShapeDtypeStruct((1024 * NUM_CAND,), jnp.float32),
                  jax.ShapeDtypeStruct((1024,), jnp.float32)),
        mesh=mesh,
        compiler_params=pltpu.CompilerParams(use_tc_tiling_on_sc=False,
                                             needs_layout_passes=False),
        scratch_types=[
            pltpu.VMEM((NUM_CAND,), jnp.int32),
            pltpu.VMEM((GSUB,), jnp.int32),
            pltpu.VMEM((GSUB, FD), jnp.float32),
            pltpu.VMEM((SB, FD), jnp.float32),
            pltpu.VMEM((NUM_CAND,), jnp.float32),
            pltpu.VMEM((SB,), jnp.int32),
            pltpu.VMEM((SB, FD), jnp.float32),
            pltpu.VMEM((16,), jnp.float32),
            pltpu.SemaphoreType.DMA,
        ],
    )(_score_kernel)
    return kfn(all_i, batch_u, cand_flat, pos_iids)


def _loss_kernel(s_ref, p_ref, o_ref):
    s = s_ref[...]
    p = p_ref[...]
    v = jax.lax.bitcast_convert_type(s, jnp.uint32)
    sign = (v >> 31).astype(jnp.uint32)
    key = v ^ jnp.where(sign == 1, jnp.uint32(0xFFFFFFFF),
                        jnp.uint32(0x80000000))

    def bit_step(i, lo):
        cand = lo | (jnp.uint32(1) << (31 - i))
        cnt = jnp.sum((key >= cand[:, None]).astype(jnp.int32), axis=1)
        return jnp.where(cnt >= NEG_PER_POS, cand, lo)

    lo = lax.fori_loop(0, 32, bit_step,
                       jnp.zeros((s.shape[0],), jnp.uint32))
    t = lo[:, None]
    gt = key > t
    c_gt = jnp.sum(gt.astype(jnp.int32), axis=1, keepdims=True)
    s_t = jnp.max(jnp.where(key <= t, s, -jnp.inf), axis=1, keepdims=True)
    smax = jnp.max(s, axis=1, keepdims=True)
    pli = p / TEMPERATURE
    m = jnp.maximum(pli, smax / TEMPERATURE)
    sumexp = (jnp.exp(pli - m)
              + jnp.sum(jnp.where(gt, jnp.exp(s / TEMPERATURE - m), 0.0),
                        axis=1, keepdims=True)
              + (NEG_PER_POS - c_gt).astype(jnp.float32)
              * jnp.exp(s_t / TEMPERATURE - m))
    lse = m + jnp.log(sumexp)
    o_ref[...] = jnp.mean(lse - pli).reshape(1, 1)


def _loss_call(scores, pos_scores):
    return pl.pallas_call(
        _loss_kernel,
        in_specs=[pl.BlockSpec((1024, NUM_CAND), lambda: (0, 0)),
                  pl.BlockSpec((1024, 1), lambda: (0, 0))],
        out_specs=pl.BlockSpec((1, 1), lambda: (0, 0)),
        out_shape=jax.ShapeDtypeStruct((1, 1), jnp.float32),
    )(scores, pos_scores)


def _normalize(x, axis=-1):
    n = jnp.linalg.norm(x, axis=axis, keepdims=True)
    return x / jnp.maximum(n, 1e-12)


def _sem_matmul_norm_block(x_ref, w_ref, b_ref, o_ref):
    y = jnp.dot(x_ref[...], w_ref[...], preferred_element_type=jnp.float32)
    y = y + b_ref[...]
    n = jnp.sqrt(jnp.sum(y * y, axis=-1, keepdims=True))
    o_ref[...] = y / jnp.maximum(n, 1e-12)


def _sem_matmul_norm(x, W, b):
    M = x.shape[0]
    BM = 2000
    assert M % BM == 0
    return pl.pallas_call(
        _sem_matmul_norm_block,
        grid=(M // BM,),
        in_specs=[
            pl.BlockSpec((BM, IN_DIM), lambda i: (i, 0)),
            pl.BlockSpec((IN_DIM, BRANCH_DIM), lambda i: (0, 0)),
            pl.BlockSpec((1, BRANCH_DIM), lambda i: (0, 0)),
        ],
        out_specs=pl.BlockSpec((BM, BRANCH_DIM), lambda i: (i, 0)),
        out_shape=jax.ShapeDtypeStruct((M, BRANCH_DIM), jnp.float32),
    )(x, W, b)


def kernel(raw_item_embs, user_sem_base, sem_W, sem_b, collab_user_emb,
           collab_item_emb, adj_val, adj_row, adj_col, uids, pos_iids,
           neg_per_pos, hard_neg_factor):
    b2d = sem_b.reshape(1, BRANCH_DIM)
    i_sem = _sem_matmul_norm(raw_item_embs, sem_W, b2d)
    u_sem_batch = _normalize(user_sem_base[uids] @ sem_W + sem_b)

    ego = jnp.concatenate([collab_user_emb, collab_item_emb], axis=0)
    ego1 = _gcn_layer(ego, adj_row, adj_col, adj_val)
    ego2 = _gcn_layer(ego1, adj_row, adj_col, adj_val)
    final = (ego + ego1 + ego2) / 3.0
    u_collab_batch = _normalize(final[uids])
    i_collab = _normalize(final[N_USERS:])

    all_i = _normalize(jnp.concatenate([i_collab, i_sem], axis=1))
    batch_u = _normalize(jnp.concatenate([u_collab_batch, u_sem_batch], axis=1))

    B = uids.shape[0]
    cand_ids = jax.random.randint(jax.random.key(42), (B, NUM_CAND), 0, N_ITEMS,
                                  dtype=jnp.int32)
    cand_ids = jnp.where(cand_ids == pos_iids[:, None],
                         (cand_ids + 1) % N_ITEMS, cand_ids)
    scores_flat, pos_scores = _score_call(all_i, batch_u,
                                          cand_ids.reshape(-1), pos_iids)
    scores = scores_flat.reshape(B, NUM_CAND)
    loss = _loss_call(scores, pos_scores.reshape(B, 1)).reshape(())
    return loss + 0.0 * (neg_per_pos * hard_neg_factor - NUM_CAND)
